# baseline (speedup 1.0000x reference)
import jax
import jax.numpy as jnp
from jax.experimental import pallas as pl
from jax.experimental.pallas import tpu as pltpu

_KC = 256


def _support_body(a_ref, x_ref, k_ref, o_ref, acc_ref):
    n = a_ref.shape[2]
    acc_ref[...] = jnp.dot(a_ref[0, :, 0:_KC], x_ref[0, 0:_KC, :],
                           preferred_element_type=jnp.float32)
    for kc in range(1, n // _KC):
        acc_ref[...] = acc_ref[...] + jnp.dot(
            a_ref[0, :, kc * _KC:(kc + 1) * _KC],
            x_ref[0, kc * _KC:(kc + 1) * _KC, :],
            preferred_element_type=jnp.float32)
    o_ref[0] = jnp.dot(acc_ref[...], k_ref[...],
                       preferred_element_type=jnp.float32)


def kernel(Xs, As, attn_kernel):
    B, N, F = Xs.shape
    K = N // 2
    BLK = 512
    support = pl.pallas_call(
        _support_body,
        grid=(B, N // BLK),
        in_specs=[
            pl.BlockSpec((1, BLK, N), lambda b, i: (b, i, 0)),
            pl.BlockSpec((1, N, F), lambda b, i: (b, 0, 0)),
            pl.BlockSpec((F, 1), lambda b, i: (0, 0)),
        ],
        out_specs=pl.BlockSpec((1, BLK, 1), lambda b, i: (b, i, 0)),
        out_shape=jax.ShapeDtypeStruct((B, N, 1), jnp.float32),
        scratch_shapes=[pltpu.VMEM((BLK, F), jnp.float32)],
    )(As, Xs, attn_kernel)
    scoring = jax.nn.softmax(support, axis=1).reshape(B, N)
    keep_values, keep_indices = jax.lax.top_k(scoring, K)
    Xs_out = jnp.take_along_axis(Xs, keep_indices[:, :, None], axis=1)
    A_rows = jnp.take_along_axis(As, keep_indices[:, :, None], axis=1)
    As_out = jnp.take_along_axis(A_rows, keep_indices[:, None, :], axis=2)
    return (Xs_out, As_out, keep_values)
